# baseline (speedup 1.0000x reference)
import functools

import jax
import jax.numpy as jnp
from jax import lax
from jax.experimental import pallas as pl
from jax.experimental.pallas import tpu as pltpu
from jax.experimental.pallas import tpu_sc as plsc

NC = 2
NS = 16
L = 16
SPAN = 128
CHUNK = 128
NBUF = 8
CAP = 16


def _make_sweep_kernel(batch: int, n_factors: int, n_rows: int):
  n_full = (n_rows // SPAN) * SPAN
  tail = n_rows - n_full
  max_off = n_full - SPAN
  base_chunks = n_full // CHUNK // NS
  rows_per_tec = base_chunks * CHUNK
  last_extra = n_full // CHUNK - base_chunks * NS
  nch = base_chunks + last_extra + 1
  n_groups = batch // L
  mesh = plsc.VectorSubcoreMesh(
      core_axis_name="c", subcore_axis_name="s", num_cores=NC, num_subcores=NS)

  @functools.partial(
      pl.kernel,
      out_type=(jax.ShapeDtypeStruct((batch, n_factors), jnp.float32),
                jax.ShapeDtypeStruct((batch, n_factors), jnp.float32)),
      mesh=mesh,
      compiler_params=pltpu.CompilerParams(needs_layout_passes=False),
      scratch_types=dict(
          pairs=pltpu.VMEM((2 * batch,), jnp.int32),
          lid=pltpu.VMEM((batch + L,), jnp.int32),
          lpd=pltpu.VMEM((batch + L,), jnp.int32),
          bid=pltpu.VMEM((nch * CAP + L,), jnp.int32),
          bpd=pltpu.VMEM((nch * CAP + L,), jnp.int32),
          counts=pltpu.SMEM((nch,), jnp.int32),
          wins=pltpu.VMEM((NBUF, n_factors, CHUNK), jnp.float32),
          ovwin=pltpu.VMEM((n_factors, SPAN), jnp.float32),
          rowbuf=pltpu.VMEM((CAP, n_factors), jnp.float32),
          tails=pltpu.VMEM((2 * tail * n_factors,), jnp.float32),
          sems=pltpu.SemaphoreType.DMA((NBUF,)),
          sem_w=pltpu.SemaphoreType.DMA,
      ),
  )
  def sweep(data_hbm, uft_hbm, ift_hbm, utail_hbm, itail_hbm, urows_hbm,
            irows_hbm, *, pairs, lid, lpd, bid, bpd, counts, wins, ovwin,
            rowbuf, tails, sems, sem_w):
    c = lax.axis_index("c")
    t = lax.axis_index("s")
    lo = t * rows_per_tec
    is_last = t == NS - 1
    hi = jnp.where(is_last, n_rows, lo + rows_per_tec)
    n_sweep = jnp.where(is_last, base_chunks + last_extra, base_chunks)

    pltpu.sync_copy(data_hbm, pairs)
    pltpu.sync_copy(utail_hbm, tails.at[pl.ds(0, tail * n_factors)])
    pltpu.sync_copy(itail_hbm,
                    tails.at[pl.ds(tail * n_factors, tail * n_factors)])
    lanes = lax.iota(jnp.int32, L)
    lane0 = lanes < 1

    def zero_counts(i, carry):
      counts[i] = 0
      return carry

    lax.fori_loop(0, nch, zero_counts, 0)

    def filt(g, cnt):
      ids = plsc.load_gather(pairs, [(g * L + lanes) * 2 + c])
      m = (ids >= lo) & (ids < hi)
      plsc.store_compressed(lid.at[pl.ds(cnt, L)], ids, mask=m)
      plsc.store_compressed(lpd.at[pl.ds(cnt, L)], g * L + lanes, mask=m)
      return cnt + plsc.all_reduce_population_count(m)[0]

    cnt = lax.fori_loop(0, n_groups, filt, 0)

    def extract_row(win, pre, id_s, r_s):
      r = jnp.full((L,), r_s, jnp.int32)
      g0 = plsc.load_gather(win, pre + [lanes, r])
      g1 = plsc.load_gather(win, pre + [lanes + L, r])
      tb = (jnp.maximum(id_s - n_full, 0) * n_factors
            + c * (tail * n_factors))
      t0 = plsc.load_gather(tails, [tb + lanes])
      t1 = plsc.load_gather(tails, [tb + L + lanes])
      in_tail = jnp.full((L,), id_s >= n_full, jnp.bool_)
      return jnp.where(in_tail, t0, g0), jnp.where(in_tail, t1, g1)

    def put_row(slot, r0, r1):
      s = jnp.full((L,), slot, jnp.int32)
      plsc.store_scatter(rowbuf, [s, lanes], r0)
      plsc.store_scatter(rowbuf, [s, lanes + L], r1)

    def run(tbl, out_hbm):
      def bucketize(h, carry):
        id_s = lid[pl.ds(h, L)][0]
        p_s = lpd[pl.ds(h, L)][0]
        ch = (id_s - lo) // CHUNK
        slot = counts[ch]
        counts[ch] = slot + 1

        @pl.when(slot < CAP)
        def _():
          pos = jnp.full((L,), ch * CAP + slot, jnp.int32)
          plsc.store_scatter(bid, [pos], jnp.full((L,), id_s, jnp.int32),
                             mask=lane0)
          plsc.store_scatter(bpd, [pos], jnp.full((L,), p_s, jnp.int32),
                             mask=lane0)

        @pl.when(slot >= CAP)
        def _():
          off = pl.multiple_of(
              jnp.minimum((id_s // SPAN) * SPAN, max_off), SPAN)
          pltpu.sync_copy(tbl.at[:, pl.ds(off, SPAN)], ovwin)
          r0, r1 = extract_row(ovwin, [], id_s, id_s % SPAN)
          put_row(0, r0, r1)
          pltpu.sync_copy(rowbuf.at[0], out_hbm.at[p_s])

        return carry

      lax.fori_loop(0, cnt, bucketize, 0)

      def fire(ck, buf):
        off = pl.multiple_of(lo + ck * CHUNK, SPAN)
        pltpu.async_copy(tbl.at[:, pl.ds(off, CHUNK)], wins.at[buf],
                         sems.at[buf])

      for j in range(NBUF - 1):

        @pl.when(j < n_sweep)
        def _(j=j):
          fire(j, j)

      def chunk_step(ck, carry):
        buf = ck % NBUF

        @pl.when(ck + NBUF - 1 < n_sweep)
        def _():
          fire(ck + NBUF - 1, (ck + NBUF - 1) % NBUF)

        pltpu.make_async_copy(tbl.at[:, pl.ds(0, CHUNK)], wins.at[buf],
                              sems.at[buf]).wait()
        nh = jnp.minimum(counts[ck], CAP)

        def hit(h, carry2):
          id_s = bid[pl.ds(ck * CAP + h, L)][0]
          p_s = bpd[pl.ds(ck * CAP + h, L)][0]
          r0, r1 = extract_row(wins, [jnp.full((L,), buf, jnp.int32)], id_s,
                               id_s - (lo + ck * CHUNK))
          put_row(h, r0, r1)
          pltpu.async_copy(rowbuf.at[h], out_hbm.at[p_s], sem_w)
          return carry2

        lax.fori_loop(0, nh, hit, 0)

        def drain(h, carry2):
          pltpu.make_async_copy(rowbuf.at[0], out_hbm.at[0], sem_w).wait()
          return carry2

        lax.fori_loop(0, nh, drain, 0)
        return carry

      lax.fori_loop(0, n_sweep, chunk_step, 0)

      @pl.when(is_last)
      def _():
        tch = nch - 1
        nh = jnp.minimum(counts[tch], CAP)

        def thit(h, carry2):
          id_s = bid[pl.ds(tch * CAP + h, L)][0]
          p_s = bpd[pl.ds(tch * CAP + h, L)][0]
          r0, r1 = extract_row(ovwin, [], id_s, 0)
          put_row(h, r0, r1)
          pltpu.async_copy(rowbuf.at[h], out_hbm.at[p_s], sem_w)
          return carry2

        lax.fori_loop(0, nh, thit, 0)

        def tdrain(h, carry2):
          pltpu.make_async_copy(rowbuf.at[0], out_hbm.at[0], sem_w).wait()
          return carry2

        lax.fori_loop(0, nh, tdrain, 0)

    @pl.when(c == 0)
    def _():
      run(uft_hbm, urows_hbm)

    @pl.when(c == 1)
    def _():
      run(ift_hbm, irows_hbm)

  return sweep


def _make_dot_kernel(batch: int, n_factors: int):
  bpw = batch // (NC * NS)
  mesh = plsc.VectorSubcoreMesh(
      core_axis_name="c", subcore_axis_name="s", num_cores=NC, num_subcores=NS)

  @functools.partial(
      pl.kernel,
      out_type=jax.ShapeDtypeStruct((batch,), jnp.float32),
      mesh=mesh,
      compiler_params=pltpu.CompilerParams(needs_layout_passes=False),
      scratch_types=dict(
          ubuf=pltpu.VMEM((bpw * n_factors,), jnp.float32),
          vbuf=pltpu.VMEM((bpw * n_factors,), jnp.float32),
          outv=pltpu.VMEM((bpw,), jnp.float32),
      ),
  )
  def dot(u_hbm, v_hbm, out_hbm, *, ubuf, vbuf, outv):
    wid = lax.axis_index("s") * NC + lax.axis_index("c")
    base = wid * bpw
    pltpu.sync_copy(u_hbm.at[pl.ds(base * n_factors, bpw * n_factors)], ubuf)
    pltpu.sync_copy(v_hbm.at[pl.ds(base * n_factors, bpw * n_factors)], vbuf)
    lanes = lax.iota(jnp.int32, L)

    def blk(b, carry):
      rows = b * L + lanes
      acc = jnp.zeros((L,), jnp.float32)
      for cc in range(n_factors):
        u = plsc.load_gather(ubuf, [rows * n_factors + cc])
        v = plsc.load_gather(vbuf, [rows * n_factors + cc])
        acc = acc + u * v
      outv[pl.ds(b * L, L)] = acc
      return carry

    lax.fori_loop(0, bpw // L, blk, 0)
    pltpu.sync_copy(outv, out_hbm.at[pl.ds(base, bpw)])

  return dot


def kernel(data, user_factors, item_factors):
  batch, _ = data.shape
  n_rows, n_factors = user_factors.shape
  n_full = (n_rows // SPAN) * SPAN
  sweep = _make_sweep_kernel(batch, n_factors, n_rows)
  urows, irows = sweep(data.reshape(-1), user_factors.T, item_factors.T,
                       user_factors[n_full:].reshape(-1),
                       item_factors[n_full:].reshape(-1))
  return _make_dot_kernel(batch, n_factors)(urows.reshape(-1),
                                            irows.reshape(-1))


# speedup vs baseline: 1.0528x; 1.0528x over previous
import functools

import jax
import jax.numpy as jnp
from jax import lax
from jax.experimental import pallas as pl
from jax.experimental.pallas import tpu as pltpu
from jax.experimental.pallas import tpu_sc as plsc

NC = 2
NS = 16
L = 16
SPAN = 128
CHUNK = 128
NBUF = 12
CAP = 16
PBLK = 8192


def _make_sweep_kernel(batch: int, n_factors: int, n_rows: int):
  n_full = (n_rows // SPAN) * SPAN
  tail = n_rows - n_full
  max_off = n_full - SPAN
  base_chunks = n_full // CHUNK // NS
  rows_per_tec = base_chunks * CHUNK
  last_extra = n_full // CHUNK - base_chunks * NS
  nch = base_chunks + last_extra + 1
  n_groups = batch // L
  mesh = plsc.VectorSubcoreMesh(
      core_axis_name="c", subcore_axis_name="s", num_cores=NC, num_subcores=NS)

  @functools.partial(
      pl.kernel,
      out_type=(jax.ShapeDtypeStruct((batch, n_factors), jnp.float32),
                jax.ShapeDtypeStruct((batch, n_factors), jnp.float32)),
      mesh=mesh,
      compiler_params=pltpu.CompilerParams(needs_layout_passes=False),
      scratch_types=dict(
          pbuf=pltpu.VMEM((PBLK,), jnp.int32),
          lid=pltpu.VMEM((batch + L,), jnp.int32),
          lpd=pltpu.VMEM((batch + L,), jnp.int32),
          bid=pltpu.VMEM((nch * CAP + L,), jnp.int32),
          bpd=pltpu.VMEM((nch * CAP + L,), jnp.int32),
          counts=pltpu.SMEM((nch,), jnp.int32),
          wins=pltpu.VMEM((NBUF, n_factors, CHUNK), jnp.float32),
          ovwin=pltpu.VMEM((n_factors, SPAN), jnp.float32),
          rowbuf=pltpu.VMEM((CAP, n_factors), jnp.float32),
          tails=pltpu.VMEM((2 * tail * n_factors,), jnp.float32),
          sems=pltpu.SemaphoreType.DMA((NBUF,)),
          sem_w=pltpu.SemaphoreType.DMA,
      ),
  )
  def sweep(data_hbm, uft_hbm, ift_hbm, utail_hbm, itail_hbm, urows_hbm,
            irows_hbm, *, pbuf, lid, lpd, bid, bpd, counts, wins, ovwin,
            rowbuf, tails, sems, sem_w):
    c = lax.axis_index("c")
    t = lax.axis_index("s")
    lo = t * rows_per_tec
    is_last = t == NS - 1
    hi = jnp.where(is_last, n_rows, lo + rows_per_tec)
    n_sweep = jnp.where(is_last, base_chunks + last_extra, base_chunks)

    pltpu.sync_copy(utail_hbm, tails.at[pl.ds(0, tail * n_factors)])
    pltpu.sync_copy(itail_hbm,
                    tails.at[pl.ds(tail * n_factors, tail * n_factors)])
    lanes = lax.iota(jnp.int32, L)
    lane0 = lanes < 1

    def zero_counts(i, carry):
      counts[i] = 0
      return carry

    lax.fori_loop(0, nch, zero_counts, 0)

    def extract_row(win, pre, id_s, r_s):
      r = jnp.full((L,), r_s, jnp.int32)
      g0 = plsc.load_gather(win, pre + [lanes, r])
      g1 = plsc.load_gather(win, pre + [lanes + L, r])
      tb = (jnp.maximum(id_s - n_full, 0) * n_factors
            + c * (tail * n_factors))
      t0 = plsc.load_gather(tails, [tb + lanes])
      t1 = plsc.load_gather(tails, [tb + L + lanes])
      in_tail = jnp.full((L,), id_s >= n_full, jnp.bool_)
      return jnp.where(in_tail, t0, g0), jnp.where(in_tail, t1, g1)

    def put_row(slot, r0, r1):
      s = jnp.full((L,), slot, jnp.int32)
      plsc.store_scatter(rowbuf, [s, lanes], r0)
      plsc.store_scatter(rowbuf, [s, lanes + L], r1)

    def run(tbl, out_hbm):
      def fire(ck, buf):
        off = pl.multiple_of(lo + ck * CHUNK, SPAN)
        pltpu.async_copy(tbl.at[:, pl.ds(off, CHUNK)], wins.at[buf],
                         sems.at[buf])

      for j in range(NBUF - 1):

        @pl.when(j < n_sweep)
        def _(j=j):
          fire(j, j)

      def ablk(blk, cnt0):
        pltpu.sync_copy(data_hbm.at[pl.ds(blk * PBLK, PBLK)], pbuf)

        def filt(g, cnt2):
          ids = plsc.load_gather(pbuf, [(g * L + lanes) * 2 + c])
          m = (ids >= lo) & (ids < hi)
          plsc.store_compressed(lid.at[pl.ds(cnt2, L)], ids, mask=m)
          plsc.store_compressed(lpd.at[pl.ds(cnt2, L)],
                                blk * (PBLK // 2) + g * L + lanes, mask=m)
          return cnt2 + plsc.all_reduce_population_count(m)[0]

        return lax.fori_loop(0, PBLK // 2 // L, filt, cnt0)

      cnt = lax.fori_loop(0, 2 * batch // PBLK, ablk, 0)

      def bucketize(h, carry):
        id_s = lid[pl.ds(h, L)][0]
        p_s = lpd[pl.ds(h, L)][0]
        ch = (id_s - lo) // CHUNK
        slot = counts[ch]
        counts[ch] = slot + 1

        @pl.when(slot < CAP)
        def _():
          pos = jnp.full((L,), ch * CAP + slot, jnp.int32)
          plsc.store_scatter(bid, [pos], jnp.full((L,), id_s, jnp.int32),
                             mask=lane0)
          plsc.store_scatter(bpd, [pos], jnp.full((L,), p_s, jnp.int32),
                             mask=lane0)

        @pl.when(slot >= CAP)
        def _():
          off = pl.multiple_of(
              jnp.minimum((id_s // SPAN) * SPAN, max_off), SPAN)
          pltpu.sync_copy(tbl.at[:, pl.ds(off, SPAN)], ovwin)
          r0, r1 = extract_row(ovwin, [], id_s, id_s % SPAN)
          put_row(0, r0, r1)
          pltpu.sync_copy(rowbuf.at[0], out_hbm.at[p_s])

        return carry

      lax.fori_loop(0, cnt, bucketize, 0)

      def chunk_step(ck, carry):
        buf = ck % NBUF

        @pl.when(ck + NBUF - 1 < n_sweep)
        def _():
          fire(ck + NBUF - 1, (ck + NBUF - 1) % NBUF)

        pltpu.make_async_copy(tbl.at[:, pl.ds(0, CHUNK)], wins.at[buf],
                              sems.at[buf]).wait()
        nh = jnp.minimum(counts[ck], CAP)

        def hit(h, carry2):
          id_s = bid[pl.ds(ck * CAP + h, L)][0]
          p_s = bpd[pl.ds(ck * CAP + h, L)][0]
          r0, r1 = extract_row(wins, [jnp.full((L,), buf, jnp.int32)], id_s,
                               id_s - (lo + ck * CHUNK))
          put_row(h, r0, r1)
          pltpu.async_copy(rowbuf.at[h], out_hbm.at[p_s], sem_w)
          return carry2

        lax.fori_loop(0, nh, hit, 0)

        def drain(h, carry2):
          pltpu.make_async_copy(rowbuf.at[0], out_hbm.at[0], sem_w).wait()
          return carry2

        lax.fori_loop(0, nh, drain, 0)
        return carry

      lax.fori_loop(0, n_sweep, chunk_step, 0)

      @pl.when(is_last)
      def _():
        tch = nch - 1
        nh = jnp.minimum(counts[tch], CAP)

        def thit(h, carry2):
          id_s = bid[pl.ds(tch * CAP + h, L)][0]
          p_s = bpd[pl.ds(tch * CAP + h, L)][0]
          r0, r1 = extract_row(ovwin, [], id_s, 0)
          put_row(h, r0, r1)
          pltpu.async_copy(rowbuf.at[h], out_hbm.at[p_s], sem_w)
          return carry2

        lax.fori_loop(0, nh, thit, 0)

        def tdrain(h, carry2):
          pltpu.make_async_copy(rowbuf.at[0], out_hbm.at[0], sem_w).wait()
          return carry2

        lax.fori_loop(0, nh, tdrain, 0)

    @pl.when(c == 0)
    def _():
      run(uft_hbm, urows_hbm)

    @pl.when(c == 1)
    def _():
      run(ift_hbm, irows_hbm)

  return sweep


def _make_dot_kernel(batch: int, n_factors: int, blk: int = 512):
  def body(u_ref, v_ref, o_ref):
    o_ref[...] = jnp.sum(u_ref[...] * v_ref[...], axis=1)

  return pl.pallas_call(
      body,
      grid=(batch // blk,),
      in_specs=[pl.BlockSpec((blk, n_factors), lambda i: (i, 0))] * 2,
      out_specs=pl.BlockSpec((blk,), lambda i: (i,)),
      out_shape=jax.ShapeDtypeStruct((batch,), jnp.float32),
  )


def kernel(data, user_factors, item_factors):
  batch, _ = data.shape
  n_rows, n_factors = user_factors.shape
  n_full = (n_rows // SPAN) * SPAN
  sweep = _make_sweep_kernel(batch, n_factors, n_rows)
  urows, irows = sweep(data.reshape(-1), user_factors.T, item_factors.T,
                       user_factors[n_full:].reshape(-1),
                       item_factors[n_full:].reshape(-1))
  return _make_dot_kernel(batch, n_factors)(urows, irows)


# speedup vs baseline: 1.0831x; 1.0288x over previous
import functools

import jax
import jax.numpy as jnp
from jax import lax
from jax.experimental import pallas as pl
from jax.experimental.pallas import tpu as pltpu
from jax.experimental.pallas import tpu_sc as plsc

NC = 2
NS = 16
L = 16
SPAN = 128
CHUNK = 256
NBUF = 6
CAP = 16
PBLK = 8192


def _make_sweep_kernel(batch: int, n_factors: int, n_rows: int):
  n_full = (n_rows // SPAN) * SPAN
  tail = n_rows - n_full
  max_off = n_full - SPAN
  base_chunks = n_full // CHUNK // NS
  rows_per_tec = base_chunks * CHUNK
  last_extra = n_full // CHUNK - base_chunks * NS
  nch = base_chunks + last_extra + 1
  n_groups = batch // L
  mesh = plsc.VectorSubcoreMesh(
      core_axis_name="c", subcore_axis_name="s", num_cores=NC, num_subcores=NS)

  @functools.partial(
      pl.kernel,
      out_type=(jax.ShapeDtypeStruct((batch, n_factors), jnp.float32),
                jax.ShapeDtypeStruct((batch, n_factors), jnp.float32)),
      mesh=mesh,
      compiler_params=pltpu.CompilerParams(needs_layout_passes=False),
      scratch_types=dict(
          pbuf=pltpu.VMEM((PBLK,), jnp.int32),
          lid=pltpu.VMEM((batch + L,), jnp.int32),
          lpd=pltpu.VMEM((batch + L,), jnp.int32),
          bid=pltpu.VMEM((nch * CAP + L,), jnp.int32),
          bpd=pltpu.VMEM((nch * CAP + L,), jnp.int32),
          counts=pltpu.SMEM((nch,), jnp.int32),
          wins=pltpu.VMEM((NBUF, n_factors, CHUNK), jnp.float32),
          ovwin=pltpu.VMEM((n_factors, SPAN), jnp.float32),
          rowbuf=pltpu.VMEM((CAP, n_factors), jnp.float32),
          tails=pltpu.VMEM((2 * tail * n_factors,), jnp.float32),
          sems=pltpu.SemaphoreType.DMA((NBUF,)),
          sem_w=pltpu.SemaphoreType.DMA,
      ),
  )
  def sweep(data_hbm, uft_hbm, ift_hbm, utail_hbm, itail_hbm, urows_hbm,
            irows_hbm, *, pbuf, lid, lpd, bid, bpd, counts, wins, ovwin,
            rowbuf, tails, sems, sem_w):
    c = lax.axis_index("c")
    t = lax.axis_index("s")
    lo = t * rows_per_tec
    is_last = t == NS - 1
    hi = jnp.where(is_last, n_rows, lo + rows_per_tec)
    n_sweep = jnp.where(is_last, base_chunks + last_extra, base_chunks)

    pltpu.sync_copy(utail_hbm, tails.at[pl.ds(0, tail * n_factors)])
    pltpu.sync_copy(itail_hbm,
                    tails.at[pl.ds(tail * n_factors, tail * n_factors)])
    lanes = lax.iota(jnp.int32, L)
    lane0 = lanes < 1

    def zero_counts(i, carry):
      counts[i] = 0
      return carry

    lax.fori_loop(0, nch, zero_counts, 0)

    def extract_row(win, pre, id_s, r_s):
      r = jnp.full((L,), r_s, jnp.int32)
      g0 = plsc.load_gather(win, pre + [lanes, r])
      g1 = plsc.load_gather(win, pre + [lanes + L, r])
      tb = (jnp.maximum(id_s - n_full, 0) * n_factors
            + c * (tail * n_factors))
      t0 = plsc.load_gather(tails, [tb + lanes])
      t1 = plsc.load_gather(tails, [tb + L + lanes])
      in_tail = jnp.full((L,), id_s >= n_full, jnp.bool_)
      return jnp.where(in_tail, t0, g0), jnp.where(in_tail, t1, g1)

    def put_row(slot, r0, r1):
      s = jnp.full((L,), slot, jnp.int32)
      plsc.store_scatter(rowbuf, [s, lanes], r0)
      plsc.store_scatter(rowbuf, [s, lanes + L], r1)

    def run(tbl, out_hbm):
      def fire(ck, buf):
        off = pl.multiple_of(lo + ck * CHUNK, SPAN)
        pltpu.async_copy(tbl.at[:, pl.ds(off, CHUNK)], wins.at[buf],
                         sems.at[buf])

      for j in range(NBUF - 1):

        @pl.when(j < n_sweep)
        def _(j=j):
          fire(j, j)

      def ablk(blk, cnt0):
        pltpu.sync_copy(data_hbm.at[pl.ds(blk * PBLK, PBLK)], pbuf)

        def filt(g, cnt2):
          ids = plsc.load_gather(pbuf, [(g * L + lanes) * 2 + c])
          m = (ids >= lo) & (ids < hi)
          plsc.store_compressed(lid.at[pl.ds(cnt2, L)], ids, mask=m)
          plsc.store_compressed(lpd.at[pl.ds(cnt2, L)],
                                blk * (PBLK // 2) + g * L + lanes, mask=m)
          return cnt2 + plsc.all_reduce_population_count(m)[0]

        return lax.fori_loop(0, PBLK // 2 // L, filt, cnt0)

      cnt = lax.fori_loop(0, 2 * batch // PBLK, ablk, 0)

      def bucketize(h, carry):
        id_s = lid[pl.ds(h, L)][0]
        p_s = lpd[pl.ds(h, L)][0]
        ch = (id_s - lo) // CHUNK
        slot = counts[ch]
        counts[ch] = slot + 1

        @pl.when(slot < CAP)
        def _():
          pos = jnp.full((L,), ch * CAP + slot, jnp.int32)
          plsc.store_scatter(bid, [pos], jnp.full((L,), id_s, jnp.int32),
                             mask=lane0)
          plsc.store_scatter(bpd, [pos], jnp.full((L,), p_s, jnp.int32),
                             mask=lane0)

        @pl.when(slot >= CAP)
        def _():
          off = pl.multiple_of(
              jnp.minimum((id_s // SPAN) * SPAN, max_off), SPAN)
          pltpu.sync_copy(tbl.at[:, pl.ds(off, SPAN)], ovwin)
          r0, r1 = extract_row(ovwin, [], id_s, id_s % SPAN)
          put_row(0, r0, r1)
          pltpu.sync_copy(rowbuf.at[0], out_hbm.at[p_s])

        return carry

      lax.fori_loop(0, cnt, bucketize, 0)

      def chunk_step(ck, carry):
        buf = ck % NBUF

        @pl.when(ck + NBUF - 1 < n_sweep)
        def _():
          fire(ck + NBUF - 1, (ck + NBUF - 1) % NBUF)

        pltpu.make_async_copy(tbl.at[:, pl.ds(0, CHUNK)], wins.at[buf],
                              sems.at[buf]).wait()
        nh = jnp.minimum(counts[ck], CAP)

        def hit(h, carry2):
          id_s = bid[pl.ds(ck * CAP + h, L)][0]
          p_s = bpd[pl.ds(ck * CAP + h, L)][0]
          r0, r1 = extract_row(wins, [jnp.full((L,), buf, jnp.int32)], id_s,
                               id_s - (lo + ck * CHUNK))
          put_row(h, r0, r1)
          pltpu.async_copy(rowbuf.at[h], out_hbm.at[p_s], sem_w)
          return carry2

        lax.fori_loop(0, nh, hit, 0)

        def drain(h, carry2):
          pltpu.make_async_copy(rowbuf.at[0], out_hbm.at[0], sem_w).wait()
          return carry2

        lax.fori_loop(0, nh, drain, 0)
        return carry

      lax.fori_loop(0, n_sweep, chunk_step, 0)

      @pl.when(is_last)
      def _():
        tch = nch - 1
        nh = jnp.minimum(counts[tch], CAP)

        def thit(h, carry2):
          id_s = bid[pl.ds(tch * CAP + h, L)][0]
          p_s = bpd[pl.ds(tch * CAP + h, L)][0]
          r0, r1 = extract_row(ovwin, [], id_s, 0)
          put_row(h, r0, r1)
          pltpu.async_copy(rowbuf.at[h], out_hbm.at[p_s], sem_w)
          return carry2

        lax.fori_loop(0, nh, thit, 0)

        def tdrain(h, carry2):
          pltpu.make_async_copy(rowbuf.at[0], out_hbm.at[0], sem_w).wait()
          return carry2

        lax.fori_loop(0, nh, tdrain, 0)

    @pl.when(c == 0)
    def _():
      run(uft_hbm, urows_hbm)

    @pl.when(c == 1)
    def _():
      run(ift_hbm, irows_hbm)

  return sweep


def _make_dot_kernel(batch: int, n_factors: int, blk: int = 512):
  def body(u_ref, v_ref, o_ref):
    o_ref[...] = jnp.sum(u_ref[...] * v_ref[...], axis=1)

  return pl.pallas_call(
      body,
      grid=(batch // blk,),
      in_specs=[pl.BlockSpec((blk, n_factors), lambda i: (i, 0))] * 2,
      out_specs=pl.BlockSpec((blk,), lambda i: (i,)),
      out_shape=jax.ShapeDtypeStruct((batch,), jnp.float32),
  )


def kernel(data, user_factors, item_factors):
  batch, _ = data.shape
  n_rows, n_factors = user_factors.shape
  n_full = (n_rows // SPAN) * SPAN
  sweep = _make_sweep_kernel(batch, n_factors, n_rows)
  urows, irows = sweep(data.reshape(-1), user_factors.T, item_factors.T,
                       user_factors[n_full:].reshape(-1),
                       item_factors[n_full:].reshape(-1))
  return _make_dot_kernel(batch, n_factors)(urows, irows)
